# Initial kernel scaffold; baseline (speedup 1.0000x reference)
import functools

import jax
import jax.numpy as jnp
from jax import lax
from jax.experimental import pallas as pl
from jax.experimental.pallas import tpu as pltpu
from jax.experimental.pallas import tpu_sc as plsc

N_NODES = 10000
N_EDGES = 320000
D = 128

NC = 2
NS = 16
NW = NC * NS
EDGES_PER_TILE = N_EDGES // NW
EB = 80
NB = EDGES_PER_TILE // EB
ROWS_PER_TILE = N_NODES // NS
CW = 8


def _sc_segment_sum(h, src2, dst2, zrows, zcnt, ones8):
    mesh = plsc.VectorSubcoreMesh(
        core_axis_name="c", subcore_axis_name="s", num_cores=NC, num_subcores=NS
    )

    @functools.partial(
        pl.kernel,
        mesh=mesh,
        out_type=[
            jax.ShapeDtypeStruct((NC, N_NODES, D), jnp.float32),
            jax.ShapeDtypeStruct((NC, N_NODES, CW), jnp.float32),
        ],
        scratch_types=[
            pltpu.VMEM((NB, EB), jnp.int32),
            pltpu.VMEM((NB, EB), jnp.int32),
            pltpu.VMEM((EB, D), jnp.float32),
            pltpu.VMEM((EB, CW), jnp.float32),
            pltpu.SemaphoreType.DMA,
            pltpu.VMEM_SHARED((N_NODES, D), jnp.float32),
            pltpu.VMEM_SHARED((N_NODES, CW), jnp.float32),
        ],
    )
    def k(h_hbm, src_hbm, dst_hbm, zrows_hbm, zcnt_hbm, ones_hbm,
          acc_out, cnt_out, src_v, dst_v, rows_v, ones_v, sem, acc_sh, cnt_sh):
        cid = lax.axis_index("c")
        sid = lax.axis_index("s")
        wid = cid * NS + sid

        pltpu.sync_copy(src_hbm.at[pl.ds(wid * NB, NB)], src_v)
        pltpu.sync_copy(dst_hbm.at[pl.ds(wid * NB, NB)], dst_v)
        pltpu.sync_copy(ones_hbm, ones_v)

        r0 = sid * ROWS_PER_TILE
        pltpu.sync_copy(zrows_hbm, acc_sh.at[pl.ds(r0, ROWS_PER_TILE)])
        pltpu.sync_copy(zcnt_hbm, cnt_sh.at[pl.ds(r0, ROWS_PER_TILE)])
        plsc.subcore_barrier()

        def body(j, carry):
            pltpu.async_copy(h_hbm.at[src_v.at[j]], rows_v, sem).wait()
            pltpu.sync_copy(rows_v, acc_sh.at[dst_v.at[j]], add=True)
            pltpu.sync_copy(ones_v, cnt_sh.at[dst_v.at[j]], add=True)
            return carry

        lax.fori_loop(0, NB, body, 0)
        plsc.subcore_barrier()

        pltpu.sync_copy(acc_sh.at[pl.ds(r0, ROWS_PER_TILE)],
                        acc_out.at[cid, pl.ds(r0, ROWS_PER_TILE)])
        pltpu.sync_copy(cnt_sh.at[pl.ds(r0, ROWS_PER_TILE)],
                        cnt_out.at[cid, pl.ds(r0, ROWS_PER_TILE)])

    return k(h, src2, dst2, zrows, zcnt, ones8)


def _tc_apply(h, acc, cnt, wt, b2):
    R = 1000

    def body(h_ref, acc_ref, cnt_ref, wt_ref, b_ref, o_ref):
        hb = h_ref[...]
        s = acc_ref[0] + acc_ref[1]
        deg = cnt_ref[0, :, 0:1] + cnt_ref[1, :, 0:1]
        c = s / jnp.maximum(deg, 1.0)
        z = (
            jnp.dot(hb, wt_ref[0:D, :], preferred_element_type=jnp.float32)
            + jnp.dot(c, wt_ref[D:2 * D, :], preferred_element_type=jnp.float32)
            + b_ref[...]
        )
        n = jnp.sqrt(jnp.sum(z * z, axis=1, keepdims=True))
        z = z / jnp.maximum(n, 1e-12)
        o_ref[...] = hb + jnp.maximum(z, 0.0)

    return pl.pallas_call(
        body,
        grid=(N_NODES // R,),
        in_specs=[
            pl.BlockSpec((R, D), lambda i: (i, 0)),
            pl.BlockSpec((NC, R, D), lambda i: (0, i, 0)),
            pl.BlockSpec((NC, R, CW), lambda i: (0, i, 0)),
            pl.BlockSpec((2 * D, D), lambda i: (0, 0)),
            pl.BlockSpec((1, D), lambda i: (0, 0)),
        ],
        out_specs=pl.BlockSpec((R, D), lambda i: (i, 0)),
        out_shape=jax.ShapeDtypeStruct((N_NODES, D), jnp.float32),
    )(h, acc, cnt, wt, b2)


@jax.jit
def kernel(h, edge_index, W, b):
    ei = edge_index.astype(jnp.int32)
    src2 = ei[0].reshape(NW * NB, EB)
    dst2 = ei[1].reshape(NW * NB, EB)
    zrows = jnp.zeros((ROWS_PER_TILE, D), jnp.float32)
    zcnt = jnp.zeros((ROWS_PER_TILE, CW), jnp.float32)
    ones8 = jnp.ones((EB, CW), jnp.float32)
    acc, cnt = _sc_segment_sum(h, src2, dst2, zrows, zcnt, ones8)
    wt = W.T
    b2 = b.reshape(1, D)
    return _tc_apply(h, acc, cnt, wt, b2)


# speedup vs baseline: 5.9084x; 5.9084x over previous
import functools

import jax
import jax.numpy as jnp
from jax import lax
from jax.experimental import pallas as pl
from jax.experimental.pallas import tpu as pltpu
from jax.experimental.pallas import tpu_sc as plsc

N_NODES = 10000
N_EDGES = 320000
D = 128
HD = D // 2

NC = 2
NS = 16
EDGES_PER_TILE = N_EDGES // NS
EB = 80
NB = EDGES_PER_TILE // EB
NP = 10240
ROWS_PER_TILE = NP // NS
CW = 16


def _sc_segment_sum(h2, src3, dst3, zrows, zcnt, ones1):
    mesh = plsc.VectorSubcoreMesh(
        core_axis_name="c", subcore_axis_name="s", num_cores=NC, num_subcores=NS
    )

    @functools.partial(
        pl.kernel,
        mesh=mesh,
        compiler_params=pltpu.CompilerParams(use_tc_tiling_on_sc=False),
        out_type=[
            jax.ShapeDtypeStruct((NC, NP, HD), jnp.float32),
            jax.ShapeDtypeStruct((NP, CW), jnp.float32),
        ],
        scratch_types=[
            pltpu.VMEM((NB, EB), jnp.int32),
            pltpu.VMEM((NB, EB), jnp.int32),
            pltpu.VMEM((EB, HD), jnp.float32),
            pltpu.VMEM((EB, CW), jnp.float32),
            pltpu.SemaphoreType.DMA,
            pltpu.VMEM_SHARED((NP, HD), jnp.float32),
            pltpu.VMEM_SHARED((NP, CW), jnp.float32),
        ],
    )
    def k(h_hbm, src_hbm, dst_hbm, zrows_hbm, zcnt_hbm, ones_hbm,
          acc_out, cnt_out, src_v, dst_v, rows_v, ones_v, sem, acc_sh, cnt_sh):
        cid = lax.axis_index("c")
        sid = lax.axis_index("s")

        pltpu.sync_copy(src_hbm.at[sid], src_v)
        pltpu.sync_copy(dst_hbm.at[sid], dst_v)
        pltpu.sync_copy(ones_hbm, ones_v)

        r0 = sid * ROWS_PER_TILE
        pltpu.sync_copy(zrows_hbm, acc_sh.at[pl.ds(r0, ROWS_PER_TILE)])
        pltpu.sync_copy(zcnt_hbm, cnt_sh.at[pl.ds(r0, ROWS_PER_TILE)])
        plsc.subcore_barrier()

        hc = h_hbm.at[cid]

        def body(j, carry):
            pltpu.async_copy(hc.at[src_v.at[j]], rows_v, sem).wait()
            pltpu.sync_copy(rows_v, acc_sh.at[dst_v.at[j]], add=True)

            @pl.when(cid == 0)
            def _():
                pltpu.sync_copy(ones_v, cnt_sh.at[dst_v.at[j]], add=True)

            return carry

        lax.fori_loop(0, NB, body, 0)
        plsc.subcore_barrier()

        pltpu.sync_copy(acc_sh.at[pl.ds(r0, ROWS_PER_TILE)],
                        acc_out.at[cid, pl.ds(r0, ROWS_PER_TILE)])

        @pl.when(cid == 0)
        def _():
            pltpu.sync_copy(cnt_sh.at[pl.ds(r0, ROWS_PER_TILE)],
                            cnt_out.at[pl.ds(r0, ROWS_PER_TILE)])

    return k(h2, src3, dst3, zrows, zcnt, ones1)


def _tc_apply(h, acc, cnt, wt, b2):
    R = 1000

    def body(h_ref, acc_ref, cnt_ref, wt_ref, b_ref, o_ref):
        hb = h_ref[...]
        deg = jnp.maximum(cnt_ref[:, 0:1], 1.0)
        c0 = acc_ref[0] / deg
        c1 = acc_ref[1] / deg
        z = (
            jnp.dot(hb, wt_ref[0:D, :], preferred_element_type=jnp.float32)
            + jnp.dot(c0, wt_ref[D:D + HD, :], preferred_element_type=jnp.float32)
            + jnp.dot(c1, wt_ref[D + HD:2 * D, :],
                      preferred_element_type=jnp.float32)
            + b_ref[...]
        )
        n = jnp.sqrt(jnp.sum(z * z, axis=1, keepdims=True))
        z = z / jnp.maximum(n, 1e-12)
        o_ref[...] = hb + jnp.maximum(z, 0.0)

    return pl.pallas_call(
        body,
        grid=(N_NODES // R,),
        in_specs=[
            pl.BlockSpec((R, D), lambda i: (i, 0)),
            pl.BlockSpec((NC, R, HD), lambda i: (0, i, 0)),
            pl.BlockSpec((R, CW), lambda i: (i, 0)),
            pl.BlockSpec((2 * D, D), lambda i: (0, 0)),
            pl.BlockSpec((1, D), lambda i: (0, 0)),
        ],
        out_specs=pl.BlockSpec((R, D), lambda i: (i, 0)),
        out_shape=jax.ShapeDtypeStruct((N_NODES, D), jnp.float32),
    )(h, acc, cnt, wt, b2)


@jax.jit
def kernel(h, edge_index, W, b):
    ei = edge_index.astype(jnp.int32)
    src3 = ei[0].reshape(NS, NB, EB)
    dst3 = ei[1].reshape(NS, NB, EB)
    h2 = h.reshape(N_NODES, NC, HD).transpose(1, 0, 2)
    zrows = jnp.zeros((ROWS_PER_TILE, HD), jnp.float32)
    zcnt = jnp.zeros((ROWS_PER_TILE, CW), jnp.float32)
    ones1 = jnp.ones((EB, CW), jnp.float32)
    acc, cnt = _sc_segment_sum(h2, src3, dst3, zrows, zcnt, ones1)
    wt = W.T
    b2 = b.reshape(1, D)
    return _tc_apply(h, acc, cnt[:N_NODES], wt, b2)


# speedup vs baseline: 7.5803x; 1.2830x over previous
import functools

import jax
import jax.numpy as jnp
from jax import lax
from jax.experimental import pallas as pl
from jax.experimental.pallas import tpu as pltpu
from jax.experimental.pallas import tpu_sc as plsc

N_NODES = 10000
N_EDGES = 320000
D = 128
HD = D // 2

NC = 2
NS = 16
EDGES_PER_TILE = N_EDGES // NS
EB = 80
NB = EDGES_PER_TILE // EB
NP = 10240
ROWS_PER_TILE = NP // NS
CW = 16


def _sc_segment_sum(h2, src3, dst3, zrows, zcnt, ones1):
    mesh = plsc.VectorSubcoreMesh(
        core_axis_name="c", subcore_axis_name="s", num_cores=NC, num_subcores=NS
    )

    @functools.partial(
        pl.kernel,
        mesh=mesh,
        compiler_params=pltpu.CompilerParams(use_tc_tiling_on_sc=False),
        out_type=[
            jax.ShapeDtypeStruct((NC, NP, HD), jnp.float32),
            jax.ShapeDtypeStruct((NC, NP, CW), jnp.float32),
        ],
        scratch_types=[
            pltpu.VMEM((NB, EB), jnp.int32),
            pltpu.VMEM((NB, EB), jnp.int32),
            pltpu.VMEM((2, EB, HD), jnp.float32),
            pltpu.VMEM((EB, CW), jnp.float32),
            pltpu.SemaphoreType.DMA,
            pltpu.VMEM_SHARED((NP, HD), jnp.float32),
            pltpu.VMEM_SHARED((NP, CW), jnp.float32),
        ],
    )
    def k(h_hbm, src_hbm, dst_hbm, zrows_hbm, zcnt_hbm, ones_hbm,
          acc_out, cnt_out, src_v, dst_v, rows_v, ones_v, sem, acc_sh, cnt_sh):
        cid = lax.axis_index("c")
        sid = lax.axis_index("s")

        pltpu.sync_copy(src_hbm.at[sid], src_v)
        pltpu.sync_copy(dst_hbm.at[sid], dst_v)
        pltpu.sync_copy(ones_hbm, ones_v)

        r0 = sid * ROWS_PER_TILE
        pltpu.sync_copy(zrows_hbm, acc_sh.at[pl.ds(r0, ROWS_PER_TILE)])
        pltpu.sync_copy(zcnt_hbm, cnt_sh.at[pl.ds(r0, ROWS_PER_TILE)])
        plsc.subcore_barrier()

        hc = h_hbm.at[cid]

        pltpu.async_copy(hc.at[src_v.at[0]], rows_v.at[0], sem)

        def body(j, carry):
            a = lax.rem(j, 2)
            pltpu.make_async_copy(hc.at[src_v.at[j]], rows_v.at[a], sem).wait()

            @pl.when(j < NB - 1)
            def _():
                pltpu.async_copy(hc.at[src_v.at[j + 1]],
                                 rows_v.at[lax.rem(j + 1, 2)], sem)

            pltpu.sync_copy(rows_v.at[a], acc_sh.at[dst_v.at[j]], add=True)

            @pl.when(a == cid)
            def _():
                pltpu.sync_copy(ones_v, cnt_sh.at[dst_v.at[j]], add=True)

            return carry

        lax.fori_loop(0, NB, body, 0)
        plsc.subcore_barrier()

        pltpu.sync_copy(acc_sh.at[pl.ds(r0, ROWS_PER_TILE)],
                        acc_out.at[cid, pl.ds(r0, ROWS_PER_TILE)])
        pltpu.sync_copy(cnt_sh.at[pl.ds(r0, ROWS_PER_TILE)],
                        cnt_out.at[cid, pl.ds(r0, ROWS_PER_TILE)])

    return k(h2, src3, dst3, zrows, zcnt, ones1)


def _tc_apply(h, acc, cnt, wt, b2):
    R = 1000

    def body(h_ref, acc_ref, cnt_ref, wt_ref, b_ref, o_ref):
        hb = h_ref[...]
        deg = jnp.maximum(cnt_ref[0, :, 0:1] + cnt_ref[1, :, 0:1], 1.0)
        c0 = acc_ref[0] / deg
        c1 = acc_ref[1] / deg
        z = (
            jnp.dot(hb, wt_ref[0:D, :], preferred_element_type=jnp.float32)
            + jnp.dot(c0, wt_ref[D:D + HD, :], preferred_element_type=jnp.float32)
            + jnp.dot(c1, wt_ref[D + HD:2 * D, :],
                      preferred_element_type=jnp.float32)
            + b_ref[...]
        )
        n = jnp.sqrt(jnp.sum(z * z, axis=1, keepdims=True))
        z = z / jnp.maximum(n, 1e-12)
        o_ref[...] = hb + jnp.maximum(z, 0.0)

    return pl.pallas_call(
        body,
        grid=(N_NODES // R,),
        in_specs=[
            pl.BlockSpec((R, D), lambda i: (i, 0)),
            pl.BlockSpec((NC, R, HD), lambda i: (0, i, 0)),
            pl.BlockSpec((NC, R, CW), lambda i: (0, i, 0)),
            pl.BlockSpec((2 * D, D), lambda i: (0, 0)),
            pl.BlockSpec((1, D), lambda i: (0, 0)),
        ],
        out_specs=pl.BlockSpec((R, D), lambda i: (i, 0)),
        out_shape=jax.ShapeDtypeStruct((N_NODES, D), jnp.float32),
    )(h, acc, cnt, wt, b2)


@jax.jit
def kernel(h, edge_index, W, b):
    ei = edge_index.astype(jnp.int32)
    src3 = ei[0].reshape(NS, NB, EB)
    dst3 = ei[1].reshape(NS, NB, EB)
    h2 = h.reshape(N_NODES, NC, HD).transpose(1, 0, 2)
    zrows = jnp.zeros((ROWS_PER_TILE, HD), jnp.float32)
    zcnt = jnp.zeros((ROWS_PER_TILE, CW), jnp.float32)
    ones1 = jnp.ones((EB, CW), jnp.float32)
    acc, cnt = _sc_segment_sum(h2, src3, dst3, zrows, zcnt, ones1)
    wt = W.T
    b2 = b.reshape(1, D)
    return _tc_apply(h, acc, cnt, wt, b2)


# speedup vs baseline: 10.9366x; 1.4428x over previous
import functools

import jax
import jax.numpy as jnp
from jax import lax
from jax.experimental import pallas as pl
from jax.experimental.pallas import tpu as pltpu
from jax.experimental.pallas import tpu_sc as plsc

N_NODES = 10000
N_EDGES = 320000
D = 128
HD = D // 2

NC = 2
NS = 16
EDGES_PER_TILE = N_EDGES // NS
EB = 80
NB = EDGES_PER_TILE // EB
NP = 10240
ROWS_PER_TILE = NP // NS
CW = 16


def _sc_segment_sum(h2, src3, dst3, zrows, zcnt, ones1):
    mesh = plsc.VectorSubcoreMesh(
        core_axis_name="c", subcore_axis_name="s", num_cores=NC, num_subcores=NS
    )

    @functools.partial(
        pl.kernel,
        mesh=mesh,
        compiler_params=pltpu.CompilerParams(use_tc_tiling_on_sc=False),
        out_type=[
            jax.ShapeDtypeStruct((NC, NP, HD), jnp.float32),
            jax.ShapeDtypeStruct((NC, NP, CW), jnp.float32),
        ],
        scratch_types=[
            pltpu.VMEM((NB, EB), jnp.int32),
            pltpu.VMEM((NB, EB), jnp.int32),
            pltpu.VMEM((3, EB, HD), jnp.float32),
            pltpu.VMEM((EB, CW), jnp.float32),
            pltpu.SemaphoreType.DMA,
            pltpu.SemaphoreType.DMA,
            pltpu.SemaphoreType.DMA,
            pltpu.VMEM_SHARED((NP, HD), jnp.float32),
            pltpu.VMEM_SHARED((NP, CW), jnp.float32),
        ],
    )
    def k(h_hbm, src_hbm, dst_hbm, zrows_hbm, zcnt_hbm, ones_hbm,
          acc_out, cnt_out, src_v, dst_v, rows_v, ones_v, sem_g, sem_s, sem_c,
          acc_sh, cnt_sh):
        cid = lax.axis_index("c")
        sid = lax.axis_index("s")

        pltpu.sync_copy(src_hbm.at[sid], src_v)
        pltpu.sync_copy(dst_hbm.at[sid], dst_v)
        pltpu.sync_copy(ones_hbm, ones_v)

        r0 = sid * ROWS_PER_TILE
        pltpu.sync_copy(zrows_hbm, acc_sh.at[pl.ds(r0, ROWS_PER_TILE)])
        pltpu.sync_copy(zcnt_hbm, cnt_sh.at[pl.ds(r0, ROWS_PER_TILE)])
        plsc.subcore_barrier()

        hc = h_hbm.at[cid]

        pltpu.async_copy(hc.at[src_v.at[0]], rows_v.at[0], sem_g)

        def body(j, carry):
            a = lax.rem(j, 3)

            @pl.when(j >= 2)
            def _():
                pltpu.make_async_copy(zrows_hbm.at[pl.ds(0, EB)], rows_v.at[a],
                                      sem_s).wait()

            @pl.when(j < NB - 1)
            def _():
                pltpu.async_copy(hc.at[src_v.at[j + 1]],
                                 rows_v.at[lax.rem(j + 1, 3)], sem_g)

            pltpu.make_async_copy(hc.at[src_v.at[j]], rows_v.at[a],
                                  sem_g).wait()
            pltpu.async_copy(rows_v.at[a], acc_sh.at[dst_v.at[j]], sem_s,
                             add=True)

            @pl.when(lax.rem(j, 2) == cid)
            def _():
                pltpu.async_copy(ones_v, cnt_sh.at[dst_v.at[j]], sem_c,
                                 add=True)

            return carry

        lax.fori_loop(0, NB, body, 0)

        pltpu.make_async_copy(zrows_hbm.at[pl.ds(0, EB)], rows_v.at[0],
                              sem_s).wait()
        pltpu.make_async_copy(zrows_hbm.at[pl.ds(0, EB)], rows_v.at[1],
                              sem_s).wait()

        def drain_counts(j, carry):
            pltpu.make_async_copy(ones_hbm, ones_v, sem_c).wait()
            return carry

        lax.fori_loop(0, NB // 2, drain_counts, 0)
        plsc.subcore_barrier()

        pltpu.sync_copy(acc_sh.at[pl.ds(r0, ROWS_PER_TILE)],
                        acc_out.at[cid, pl.ds(r0, ROWS_PER_TILE)])
        pltpu.sync_copy(cnt_sh.at[pl.ds(r0, ROWS_PER_TILE)],
                        cnt_out.at[cid, pl.ds(r0, ROWS_PER_TILE)])

    return k(h2, src3, dst3, zrows, zcnt, ones1)


def _tc_apply(h, acc, cnt, wt, b2):
    R = 1000

    def body(h_ref, acc_ref, cnt_ref, wt_ref, b_ref, o_ref):
        hb = h_ref[...]
        deg = jnp.maximum(cnt_ref[0, :, 0:1] + cnt_ref[1, :, 0:1], 1.0)
        c0 = acc_ref[0] / deg
        c1 = acc_ref[1] / deg
        z = (
            jnp.dot(hb, wt_ref[0:D, :], preferred_element_type=jnp.float32)
            + jnp.dot(c0, wt_ref[D:D + HD, :], preferred_element_type=jnp.float32)
            + jnp.dot(c1, wt_ref[D + HD:2 * D, :],
                      preferred_element_type=jnp.float32)
            + b_ref[...]
        )
        n = jnp.sqrt(jnp.sum(z * z, axis=1, keepdims=True))
        z = z / jnp.maximum(n, 1e-12)
        o_ref[...] = hb + jnp.maximum(z, 0.0)

    return pl.pallas_call(
        body,
        grid=(N_NODES // R,),
        in_specs=[
            pl.BlockSpec((R, D), lambda i: (i, 0)),
            pl.BlockSpec((NC, R, HD), lambda i: (0, i, 0)),
            pl.BlockSpec((NC, R, CW), lambda i: (0, i, 0)),
            pl.BlockSpec((2 * D, D), lambda i: (0, 0)),
            pl.BlockSpec((1, D), lambda i: (0, 0)),
        ],
        out_specs=pl.BlockSpec((R, D), lambda i: (i, 0)),
        out_shape=jax.ShapeDtypeStruct((N_NODES, D), jnp.float32),
    )(h, acc, cnt, wt, b2)


@jax.jit
def kernel(h, edge_index, W, b):
    ei = edge_index.astype(jnp.int32)
    src3 = ei[0].reshape(NS, NB, EB)
    dst3 = ei[1].reshape(NS, NB, EB)
    h2 = h.reshape(N_NODES, NC, HD).transpose(1, 0, 2)
    zrows = jnp.zeros((ROWS_PER_TILE, HD), jnp.float32)
    zcnt = jnp.zeros((ROWS_PER_TILE, CW), jnp.float32)
    ones1 = jnp.ones((EB, CW), jnp.float32)
    acc, cnt = _sc_segment_sum(h2, src3, dst3, zrows, zcnt, ones1)
    wt = W.T
    b2 = b.reshape(1, D)
    return _tc_apply(h, acc, cnt, wt, b2)


# speedup vs baseline: 11.9805x; 1.0955x over previous
import functools

import jax
import jax.numpy as jnp
from jax import lax
from jax.experimental import pallas as pl
from jax.experimental.pallas import tpu as pltpu
from jax.experimental.pallas import tpu_sc as plsc

N_NODES = 10000
N_EDGES = 320000
D = 128
HD = D // 2

NC = 2
NS = 16
EDGES_PER_TILE = N_EDGES // NS
EB = 125
NB = EDGES_PER_TILE // EB
NP = 10240
ROWS_PER_TILE = NP // NS
CW = 16


def _sc_segment_sum(h2, src3, dst3, zrows, zcnt, ones1):
    mesh = plsc.VectorSubcoreMesh(
        core_axis_name="c", subcore_axis_name="s", num_cores=NC, num_subcores=NS
    )

    @functools.partial(
        pl.kernel,
        mesh=mesh,
        compiler_params=pltpu.CompilerParams(use_tc_tiling_on_sc=False),
        out_type=[
            jax.ShapeDtypeStruct((NC, NP, HD), jnp.float32),
            jax.ShapeDtypeStruct((NC, NP, CW), jnp.float32),
        ],
        scratch_types=[
            pltpu.VMEM((NB, EB), jnp.int32),
            pltpu.VMEM((NB, EB), jnp.int32),
            pltpu.VMEM((3, EB, HD), jnp.float32),
            pltpu.VMEM((EB, CW), jnp.float32),
            pltpu.SemaphoreType.DMA,
            pltpu.SemaphoreType.DMA,
            pltpu.SemaphoreType.DMA,
            pltpu.VMEM_SHARED((NP, HD), jnp.float32),
            pltpu.VMEM_SHARED((NP, CW), jnp.float32),
        ],
    )
    def k(h_hbm, src_hbm, dst_hbm, zrows_hbm, zcnt_hbm, ones_hbm,
          acc_out, cnt_out, src_v, dst_v, rows_v, ones_v, sem_g, sem_s, sem_c,
          acc_sh, cnt_sh):
        cid = lax.axis_index("c")
        sid = lax.axis_index("s")

        pltpu.sync_copy(src_hbm.at[sid], src_v)
        pltpu.sync_copy(dst_hbm.at[sid], dst_v)
        pltpu.sync_copy(ones_hbm, ones_v)

        r0 = sid * ROWS_PER_TILE
        pltpu.sync_copy(zrows_hbm, acc_sh.at[pl.ds(r0, ROWS_PER_TILE)])
        pltpu.sync_copy(zcnt_hbm, cnt_sh.at[pl.ds(r0, ROWS_PER_TILE)])
        plsc.subcore_barrier()

        hc = h_hbm.at[cid]

        pltpu.async_copy(hc.at[src_v.at[0]], rows_v.at[0], sem_g)

        def body(j, carry):
            a = lax.rem(j, 3)

            @pl.when(j >= 2)
            def _():
                pltpu.make_async_copy(zrows_hbm.at[pl.ds(0, EB)], rows_v.at[a],
                                      sem_s).wait()

            @pl.when(j < NB - 1)
            def _():
                pltpu.async_copy(hc.at[src_v.at[j + 1]],
                                 rows_v.at[lax.rem(j + 1, 3)], sem_g)

            pltpu.make_async_copy(hc.at[src_v.at[j]], rows_v.at[a],
                                  sem_g).wait()
            pltpu.async_copy(rows_v.at[a], acc_sh.at[dst_v.at[j]], sem_s,
                             add=True)

            @pl.when(lax.rem(j, 2) == cid)
            def _():
                pltpu.async_copy(ones_v, cnt_sh.at[dst_v.at[j]], sem_c,
                                 add=True)

            return carry

        lax.fori_loop(0, NB, body, 0)

        pltpu.make_async_copy(zrows_hbm.at[pl.ds(0, EB)], rows_v.at[0],
                              sem_s).wait()
        pltpu.make_async_copy(zrows_hbm.at[pl.ds(0, EB)], rows_v.at[1],
                              sem_s).wait()

        def drain_counts(j, carry):
            pltpu.make_async_copy(ones_hbm, ones_v, sem_c).wait()
            return carry

        lax.fori_loop(0, NB // 2, drain_counts, 0)
        plsc.subcore_barrier()

        pltpu.sync_copy(acc_sh.at[pl.ds(r0, ROWS_PER_TILE)],
                        acc_out.at[cid, pl.ds(r0, ROWS_PER_TILE)])
        pltpu.sync_copy(cnt_sh.at[pl.ds(r0, ROWS_PER_TILE)],
                        cnt_out.at[cid, pl.ds(r0, ROWS_PER_TILE)])

    return k(h2, src3, dst3, zrows, zcnt, ones1)


def _tc_apply(h, acc, cnt, wt, b2):
    R = 1000

    def body(h_ref, acc_ref, cnt_ref, wt_ref, b_ref, o_ref):
        hb = h_ref[...]
        deg = jnp.maximum(cnt_ref[0, :, 0:1] + cnt_ref[1, :, 0:1], 1.0)
        c0 = acc_ref[0] / deg
        c1 = acc_ref[1] / deg
        z = (
            jnp.dot(hb, wt_ref[0:D, :], preferred_element_type=jnp.float32)
            + jnp.dot(c0, wt_ref[D:D + HD, :], preferred_element_type=jnp.float32)
            + jnp.dot(c1, wt_ref[D + HD:2 * D, :],
                      preferred_element_type=jnp.float32)
            + b_ref[...]
        )
        n = jnp.sqrt(jnp.sum(z * z, axis=1, keepdims=True))
        z = z / jnp.maximum(n, 1e-12)
        o_ref[...] = hb + jnp.maximum(z, 0.0)

    return pl.pallas_call(
        body,
        grid=(N_NODES // R,),
        in_specs=[
            pl.BlockSpec((R, D), lambda i: (i, 0)),
            pl.BlockSpec((NC, R, HD), lambda i: (0, i, 0)),
            pl.BlockSpec((NC, R, CW), lambda i: (0, i, 0)),
            pl.BlockSpec((2 * D, D), lambda i: (0, 0)),
            pl.BlockSpec((1, D), lambda i: (0, 0)),
        ],
        out_specs=pl.BlockSpec((R, D), lambda i: (i, 0)),
        out_shape=jax.ShapeDtypeStruct((N_NODES, D), jnp.float32),
    )(h, acc, cnt, wt, b2)


@jax.jit
def kernel(h, edge_index, W, b):
    ei = edge_index.astype(jnp.int32)
    src3 = ei[0].reshape(NS, NB, EB)
    dst3 = ei[1].reshape(NS, NB, EB)
    h2 = h.reshape(N_NODES, NC, HD).transpose(1, 0, 2)
    zrows = jnp.zeros((ROWS_PER_TILE, HD), jnp.float32)
    zcnt = jnp.zeros((ROWS_PER_TILE, CW), jnp.float32)
    ones1 = jnp.ones((EB, CW), jnp.float32)
    acc, cnt = _sc_segment_sum(h2, src3, dst3, zrows, zcnt, ones1)
    wt = W.T
    b2 = b.reshape(1, D)
    return _tc_apply(h, acc, cnt, wt, b2)
